# baseline (speedup 1.0000x reference)
import functools

import jax
import jax.numpy as jnp
from jax import lax
from jax.experimental import pallas as pl
from jax.experimental.pallas import tpu as pltpu
from jax.experimental.pallas import tpu_sc as plsc

B = 4096
L = 200
H = 64
NC = 2
NS = 16
NW = NC * NS
BPW = B // NW
LP = 208
NG = LP // 16
LANES = 16

_mesh = plsc.VectorSubcoreMesh(
    core_axis_name="c", subcore_axis_name="s", num_cores=NC, num_subcores=NS
)


@functools.partial(
    pl.kernel,
    out_type=jax.ShapeDtypeStruct((B, L), jnp.float32),
    mesh=_mesh,
    compiler_params=pltpu.CompilerParams(
        needs_layout_passes=False, use_tc_tiling_on_sc=False
    ),
    scratch_types=[
        pltpu.VMEM((BPW, H), jnp.float32),
        pltpu.VMEM((BPW, L), jnp.int32),
        pltpu.VMEM((LP, H), jnp.float32),
        pltpu.VMEM((LP,), jnp.float32),
        pltpu.SemaphoreType.DMA,
    ],
)
def _entity_head(hid_hbm, kb_hbm, tab_hbm, out_hbm, hid_v, idx_v, rows_v, outb_v, sem):
    wid = lax.axis_index("s") * NC + lax.axis_index("c")
    b0 = wid * BPW
    pltpu.sync_copy(hid_hbm.at[pl.ds(b0, BPW)], hid_v)
    pltpu.sync_copy(kb_hbm.at[pl.ds(b0, BPW)], idx_v)

    def batch_body(bl, carry):
        b = b0 + bl
        cp1 = pltpu.async_copy(
            tab_hbm.at[idx_v.at[bl, pl.ds(0, 128)]], rows_v.at[pl.ds(0, 128)], sem
        )
        cp2 = pltpu.async_copy(
            tab_hbm.at[idx_v.at[bl, pl.ds(128, L - 128)]],
            rows_v.at[pl.ds(128, L - 128)],
            sem,
        )
        cp1.wait()
        cp2.wait()

        hv = [hid_v[bl, pl.ds(c * LANES, LANES)] for c in range(H // LANES)]
        lane15 = lax.iota(jnp.int32, LANES) == (LANES - 1)

        def tok_body(t, carry2):
            prod = rows_v[t, pl.ds(0, LANES)] * hv[0]
            for c in range(1, H // LANES):
                prod = prod + rows_v[t, pl.ds(c * LANES, LANES)] * hv[c]
            csum = plsc.cumsum(prod)
            plsc.store_scatter(
                outb_v, [jnp.full((LANES,), t, jnp.int32)], csum, mask=lane15
            )
            return carry2

        lax.fori_loop(0, L, tok_body, 0, unroll=8)
        pltpu.sync_copy(outb_v.at[pl.ds(0, L)], out_hbm.at[b])
        return carry

    lax.fori_loop(0, BPW, batch_body, 0, unroll=1)


def kernel(hidden_state, kb_arr, global_pointer, emb_table):
    del global_pointer
    kb = kb_arr.astype(jnp.int32)
    return _entity_head(hidden_state, kb, emb_table)


# speedup vs baseline: 1.1505x; 1.1505x over previous
import functools

import jax
import jax.numpy as jnp
from jax import lax
from jax.experimental import pallas as pl
from jax.experimental.pallas import tpu as pltpu
from jax.experimental.pallas import tpu_sc as plsc

B = 4096
L = 200
H = 64
NC = 2
NS = 16
NW = NC * NS
BPW = B // NW
LANES = 16
NBUF = 4
LEG = 128

_mesh = plsc.VectorSubcoreMesh(
    core_axis_name="c", subcore_axis_name="s", num_cores=NC, num_subcores=NS
)


@functools.partial(
    pl.kernel,
    out_type=jax.ShapeDtypeStruct((B, L), jnp.float32),
    mesh=_mesh,
    compiler_params=pltpu.CompilerParams(
        needs_layout_passes=False, use_tc_tiling_on_sc=False
    ),
    scratch_types=[
        pltpu.VMEM((BPW, H), jnp.float32),
        pltpu.VMEM((BPW, L), jnp.int32),
        pltpu.VMEM((NBUF, L, H), jnp.float32),
        pltpu.VMEM((NBUF, L), jnp.float32),
        pltpu.SemaphoreType.DMA,
        pltpu.SemaphoreType.DMA,
    ],
)
def _entity_head(
    hid_hbm, kb_hbm, tab_hbm, out_hbm, hid_v, idx_v, rows_v, outb_v, gsem, osem
):
    wid = lax.axis_index("s") * NC + lax.axis_index("c")
    b0 = wid * BPW
    pltpu.sync_copy(hid_hbm.at[pl.ds(b0, BPW)], hid_v)
    pltpu.sync_copy(kb_hbm.at[pl.ds(b0, BPW)], idx_v)

    def fire_gather(bl, slot):
        pltpu.async_copy(
            tab_hbm.at[idx_v.at[bl, pl.ds(0, LEG)]],
            rows_v.at[slot, pl.ds(0, LEG)],
            gsem,
        )
        pltpu.async_copy(
            tab_hbm.at[idx_v.at[bl, pl.ds(LEG, L - LEG)]],
            rows_v.at[slot, pl.ds(LEG, L - LEG)],
            gsem,
        )

    for p in range(NBUF):
        fire_gather(p, p)

    def batch_body(bl, carry):
        slot = lax.rem(bl, NBUF)
        pltpu.make_async_copy(
            tab_hbm.at[idx_v.at[bl, pl.ds(0, LEG)]],
            rows_v.at[slot, pl.ds(0, LEG)],
            gsem,
        ).wait()
        pltpu.make_async_copy(
            tab_hbm.at[idx_v.at[bl, pl.ds(LEG, L - LEG)]],
            rows_v.at[slot, pl.ds(LEG, L - LEG)],
            gsem,
        ).wait()

        @pl.when(bl >= NBUF)
        def _():
            pltpu.make_async_copy(
                outb_v.at[slot], out_hbm.at[b0 + bl - NBUF], osem
            ).wait()

        hv = [hid_v[bl, pl.ds(c * LANES, LANES)] for c in range(H // LANES)]
        lane15 = lax.iota(jnp.int32, LANES) == (LANES - 1)

        def tok_body(t, carry2):
            prod = rows_v[slot, t, pl.ds(0, LANES)] * hv[0]
            for c in range(1, H // LANES):
                prod = prod + rows_v[slot, t, pl.ds(c * LANES, LANES)] * hv[c]
            csum = plsc.cumsum(prod)
            plsc.store_scatter(
                outb_v.at[slot], [jnp.full((LANES,), t, jnp.int32)], csum, mask=lane15
            )
            return carry2

        lax.fori_loop(0, L, tok_body, 0, unroll=8)

        @pl.when(bl + NBUF < BPW)
        def _():
            fire_gather(bl + NBUF, slot)

        pltpu.async_copy(outb_v.at[slot], out_hbm.at[b0 + bl], osem)
        return carry

    lax.fori_loop(0, BPW, batch_body, 0, unroll=1)

    for p in range(NBUF):
        bl = BPW - NBUF + p
        pltpu.make_async_copy(
            outb_v.at[lax.rem(jnp.int32(bl), NBUF)], out_hbm.at[b0 + bl], osem
        ).wait()


def kernel(hidden_state, kb_arr, global_pointer, emb_table):
    del global_pointer
    kb = kb_arr.astype(jnp.int32)
    return _entity_head(hidden_state, kb, emb_table)


# speedup vs baseline: 1.6230x; 1.4107x over previous
import functools

import jax
import jax.numpy as jnp
from jax import lax
from jax.experimental import pallas as pl
from jax.experimental.pallas import tpu as pltpu
from jax.experimental.pallas import tpu_sc as plsc

B = 4096
L = 200
H = 64
NC = 2
NS = 16
NW = NC * NS
BPW = B // NW
LANES = 16
NBUF = 4
LEG = 128

_PICK_DNUMS = lax.GatherDimensionNumbers(
    offset_dims=(), collapsed_slice_dims=(0,), start_index_map=(0,)
)


def _bcast_lane(vec, lane_idx):
    return lax.gather(
        vec,
        lane_idx[:, None],
        _PICK_DNUMS,
        (1,),
        mode=lax.GatherScatterMode.PROMISE_IN_BOUNDS,
    )


_mesh = plsc.VectorSubcoreMesh(
    core_axis_name="c", subcore_axis_name="s", num_cores=NC, num_subcores=NS
)


@functools.partial(
    pl.kernel,
    out_type=jax.ShapeDtypeStruct((B, L), jnp.float32),
    mesh=_mesh,
    compiler_params=pltpu.CompilerParams(
        needs_layout_passes=False, use_tc_tiling_on_sc=False
    ),
    scratch_types=[
        pltpu.VMEM((BPW, H), jnp.float32),
        pltpu.VMEM((BPW, L), jnp.int32),
        pltpu.VMEM((NBUF, L, H), jnp.float32),
        pltpu.VMEM((NBUF, 208), jnp.float32),
        pltpu.SemaphoreType.DMA,
        pltpu.SemaphoreType.DMA,
    ],
)
def _entity_head(
    hid_hbm, kb_hbm, tab_hbm, out_hbm, hid_v, idx_v, rows_v, outb_v, gsem, osem
):
    wid = lax.axis_index("s") * NC + lax.axis_index("c")
    b0 = wid * BPW
    pltpu.sync_copy(hid_hbm.at[pl.ds(b0, BPW)], hid_v)
    pltpu.sync_copy(kb_hbm.at[pl.ds(b0, BPW)], idx_v)

    def fire_gather(bl, slot):
        pltpu.async_copy(
            tab_hbm.at[idx_v.at[bl, pl.ds(0, LEG)]],
            rows_v.at[slot, pl.ds(0, LEG)],
            gsem,
        )
        pltpu.async_copy(
            tab_hbm.at[idx_v.at[bl, pl.ds(LEG, L - LEG)]],
            rows_v.at[slot, pl.ds(LEG, L - LEG)],
            gsem,
        )

    for p in range(NBUF):
        fire_gather(p, p)

    def batch_body(bl, carry):
        slot = lax.rem(bl, NBUF)
        pltpu.make_async_copy(
            tab_hbm.at[idx_v.at[bl, pl.ds(0, LEG)]],
            rows_v.at[slot, pl.ds(0, LEG)],
            gsem,
        ).wait()
        pltpu.make_async_copy(
            tab_hbm.at[idx_v.at[bl, pl.ds(LEG, L - LEG)]],
            rows_v.at[slot, pl.ds(LEG, L - LEG)],
            gsem,
        ).wait()

        @pl.when(bl >= NBUF)
        def _():
            pltpu.make_async_copy(
                outb_v.at[slot, pl.ds(0, L)], out_hbm.at[b0 + bl - NBUF], osem
            ).wait()

        hv = [hid_v[bl, pl.ds(c * LANES, LANES)] for c in range(H // LANES)]
        lane_iota = lax.iota(jnp.int32, LANES)
        pick15 = jnp.full((LANES,), LANES - 1, jnp.int32)

        def dot16(t):
            prod = rows_v[slot, t, pl.ds(0, LANES)] * hv[0]
            for c in range(1, H // LANES):
                prod = prod + rows_v[slot, t, pl.ds(c * LANES, LANES)] * hv[c]
            csum = plsc.cumsum(prod)
            return _bcast_lane(csum, pick15)

        def blk_body(i, carry2):
            t0 = i * LANES
            res = jnp.zeros((LANES,), jnp.float32)
            for k in range(LANES):
                res = jnp.where(lane_iota == k, dot16(t0 + k), res)
            outb_v[slot, pl.ds(t0, LANES)] = res
            return carry2

        lax.fori_loop(0, L // LANES, blk_body, 0, unroll=1)

        res = jnp.zeros((LANES,), jnp.float32)
        for k in range(L - (L // LANES) * LANES):
            res = jnp.where(lane_iota == k, dot16((L // LANES) * LANES + k), res)
        outb_v[slot, pl.ds((L // LANES) * LANES, LANES)] = res

        @pl.when(bl + NBUF < BPW)
        def _():
            fire_gather(bl + NBUF, slot)

        pltpu.async_copy(outb_v.at[slot, pl.ds(0, L)], out_hbm.at[b0 + bl], osem)
        return carry

    lax.fori_loop(0, BPW, batch_body, 0, unroll=1)

    for p in range(NBUF):
        bl = BPW - NBUF + p
        pltpu.make_async_copy(
            outb_v.at[lax.rem(jnp.int32(bl), NBUF), pl.ds(0, L)],
            out_hbm.at[b0 + bl],
            osem,
        ).wait()


def kernel(hidden_state, kb_arr, global_pointer, emb_table):
    del global_pointer
    kb = kb_arr.astype(jnp.int32)
    return _entity_head(hidden_state, kb, emb_table)
